# baseline (speedup 1.0000x reference)
import functools

import jax
import jax.numpy as jnp
from jax import lax
from jax.experimental import pallas as pl
from jax.experimental.pallas import tpu as pltpu
from jax.experimental.pallas import tpu_sc as plsc

_LANES = 16


def _lane_bcast(v, k):
    idx = jnp.full((_LANES,), k, dtype=jnp.int32)
    return lax.gather(
        v, idx[:, None],
        lax.GatherDimensionNumbers(
            offset_dims=(), collapsed_slice_dims=(0,), start_index_map=(0,)),
        (1,), mode=lax.GatherScatterMode.PROMISE_IN_BOUNDS)


def _make_sc_edge_agg(B, Nn, De):
    CH = 16
    NCH = Nn // CH
    nt = Nn // _LANES
    NACC = 4
    LW = 128
    er_b = Nn * Nn * De // LW
    er_c = CH * Nn * De // LW
    ar_b = Nn * Nn // LW
    mr_b = Nn * De // LW

    mesh = plsc.VectorSubcoreMesh(core_axis_name="c", subcore_axis_name="s")

    @functools.partial(
        pl.kernel,
        out_type=jax.ShapeDtypeStruct((B * mr_b, LW), jnp.float32),
        mesh=mesh,
        scratch_types=[
            pltpu.VMEM((ar_b, LW), jnp.float32),
            pltpu.VMEM((2, er_c, LW), jnp.float32),
            pltpu.VMEM((mr_b, LW), jnp.float32),
            pltpu.SemaphoreType.DMA,
            pltpu.SemaphoreType.DMA,
            pltpu.SemaphoreType.DMA,
        ],
    )
    def sc_edge_agg(a_hbm, e_hbm, m_hbm, a_v, e_v, m_v, sem_a, sem_e0, sem_e1):
        cid = lax.axis_index("c")
        sid = lax.axis_index("s")
        w = sid * 2 + cid
        cp_a = pltpu.async_copy(a_hbm.at[pl.ds(w * ar_b, ar_b)], a_v, sem_a)
        sems = (sem_e0, sem_e1)
        cp = [None, None]
        cp[0] = pltpu.async_copy(
            e_hbm.at[pl.ds(w * er_b, er_c)], e_v.at[0], sems[0])
        cp_a.wait()
        for ch in range(NCH):
            buf = ch % 2
            if ch + 1 < NCH:
                cp[1 - buf] = pltpu.async_copy(
                    e_hbm.at[pl.ds(w * er_b + (ch + 1) * er_c, er_c)],
                    e_v.at[1 - buf], sems[1 - buf])
            cp[buf].wait()

            def row_body(i_loc, _, *, buf=buf, ch=ch):
                i = ch * CH + i_loc
                zero = jnp.zeros((De,), jnp.float32)

                def t_body(t, accs):
                    av = a_v[i, pl.ds(t * _LANES, _LANES)]
                    rbase = i_loc * (Nn * De // LW) + t * (_LANES * De // LW)
                    accs = list(accs)
                    for k in range(_LANES):
                        ev = e_v[buf, rbase + k // 8, pl.ds((k % 8) * De, De)]
                        accs[k % NACC] = (accs[k % NACC]
                                          + _lane_bcast(av, k) * ev)
                    return tuple(accs)

                accs = lax.fori_loop(0, nt, t_body, (zero,) * NACC)
                acc = (accs[0] + accs[1]) + (accs[2] + accs[3])
                m_v[i // 8, pl.ds((i % 8) * De, De)] = acc
                return 0

            lax.fori_loop(0, CH, row_body, 0)
        pltpu.sync_copy(m_v, m_hbm.at[pl.ds(w * mr_b, mr_b)])

    return sc_edge_agg


def _tc_body(h_ref, a_ref, m_ref, wcat_ref, we_ref, b_ref, o_ref, *, d):
    hw = jnp.dot(h_ref[...], wcat_ref[...],
                 preferred_element_type=jnp.float32)
    deg = jnp.sum(a_ref[...], axis=1, keepdims=True)
    me = jnp.dot(m_ref[...], we_ref[...], preferred_element_type=jnp.float32)
    o_ref[...] = hw[:, :d] + deg * hw[:, d:] + me + b_ref[...]


def kernel(H, A, E, N, W_self, W_nei, bias):
    B, Nn, D = H.shape
    De = E.shape[-1]

    sc_call = _make_sc_edge_agg(B, Nn, De)
    M = sc_call(A.reshape(-1, 128), E.reshape(-1, 128))

    H4 = H.reshape(B * Nn, D)
    A4 = A.reshape(B * Nn, Nn)
    M4 = M.reshape(B * Nn, De)
    W_cat = jnp.concatenate([W_self, W_nei[:D]], axis=1)
    W_e = W_nei[D:]
    bias2 = bias[None, :]

    RB = 512
    grid = (B * Nn // RB,)
    out = pl.pallas_call(
        functools.partial(_tc_body, d=D),
        grid=grid,
        in_specs=[
            pl.BlockSpec((RB, D), lambda r: (r, 0)),
            pl.BlockSpec((RB, Nn), lambda r: (r, 0)),
            pl.BlockSpec((RB, De), lambda r: (r, 0)),
            pl.BlockSpec((D, 2 * D), lambda r: (0, 0)),
            pl.BlockSpec((De, D), lambda r: (0, 0)),
            pl.BlockSpec((1, D), lambda r: (0, 0)),
        ],
        out_specs=pl.BlockSpec((RB, D), lambda r: (r, 0)),
        out_shape=jax.ShapeDtypeStruct((B * Nn, D), jnp.float32),
        compiler_params=pltpu.CompilerParams(
            dimension_semantics=("arbitrary",),
        ),
    )(H4, A4, M4, W_cat, W_e, bias2)
    return out.reshape(B, Nn, D)


# speedup vs baseline: 2.3054x; 2.3054x over previous
import functools

import jax
import jax.numpy as jnp
from jax import lax
from jax.experimental import pallas as pl
from jax.experimental.pallas import tpu as pltpu
from jax.experimental.pallas import tpu_sc as plsc

_LANES = 16


def _lane_bcast(v, k):
    idx = jnp.full((_LANES,), k, dtype=jnp.int32)
    return lax.gather(
        v, idx[:, None],
        lax.GatherDimensionNumbers(
            offset_dims=(), collapsed_slice_dims=(0,), start_index_map=(0,)),
        (1,), mode=lax.GatherScatterMode.PROMISE_IN_BOUNDS)


def _make_sc_edge_agg(B, Nn, De):
    CH = 16
    NCH = Nn // CH
    nt = Nn // _LANES
    NACC = 4
    rw = Nn * De

    mesh = plsc.VectorSubcoreMesh(core_axis_name="c", subcore_axis_name="s")

    @functools.partial(
        pl.kernel,
        out_type=jax.ShapeDtypeStruct((B, Nn, De), jnp.float32),
        mesh=mesh,
        scratch_types=[
            pltpu.VMEM((Nn, Nn), jnp.float32),
            pltpu.VMEM((2, CH, rw), jnp.float32),
            pltpu.VMEM((Nn, De), jnp.float32),
            pltpu.SemaphoreType.DMA,
            pltpu.SemaphoreType.DMA,
            pltpu.SemaphoreType.DMA,
        ],
    )
    def sc_edge_agg(a_hbm, e_hbm, m_hbm, a_v, e_v, m_v, sem_a, sem_e0, sem_e1):
        cid = lax.axis_index("c")
        sid = lax.axis_index("s")
        w = sid * 2 + cid
        cp_a = pltpu.async_copy(a_hbm.at[w], a_v, sem_a)
        sems = (sem_e0, sem_e1)
        cp = [None, None]
        cp[0] = pltpu.async_copy(e_hbm.at[w, pl.ds(0, CH)], e_v.at[0], sems[0])
        cp_a.wait()
        for ch in range(NCH):
            buf = ch % 2
            if ch + 1 < NCH:
                cp[1 - buf] = pltpu.async_copy(
                    e_hbm.at[w, pl.ds((ch + 1) * CH, CH)],
                    e_v.at[1 - buf], sems[1 - buf])
            cp[buf].wait()

            def row_body(i_loc, _, *, buf=buf, ch=ch):
                i = ch * CH + i_loc
                zero = jnp.zeros((De,), jnp.float32)

                def t_body(t, accs):
                    av = a_v[i, pl.ds(t * _LANES, _LANES)]
                    accs = list(accs)
                    for k in range(_LANES):
                        ev = e_v[buf, i_loc,
                                 pl.ds(t * _LANES * De + k * De, De)]
                        accs[k % NACC] = (accs[k % NACC]
                                          + _lane_bcast(av, k) * ev)
                    return tuple(accs)

                accs = lax.fori_loop(0, nt, t_body, (zero,) * NACC)
                acc = (accs[0] + accs[1]) + (accs[2] + accs[3])
                m_v[i, :] = acc
                return 0

            lax.fori_loop(0, CH, row_body, 0)
        pltpu.sync_copy(m_v, m_hbm.at[w])

    return sc_edge_agg


def _tc_body(h_ref, a_ref, m_ref, wcat_ref, we_ref, b_ref, o_ref, *, d):
    hw = jnp.dot(h_ref[...], wcat_ref[...],
                 preferred_element_type=jnp.float32)
    deg = jnp.sum(a_ref[...], axis=1, keepdims=True)
    me = jnp.dot(m_ref[...], we_ref[...], preferred_element_type=jnp.float32)
    o_ref[...] = hw[:, :d] + deg * hw[:, d:] + me + b_ref[...]


def kernel(H, A, E, N, W_self, W_nei, bias):
    B, Nn, D = H.shape
    De = E.shape[-1]

    sc_call = _make_sc_edge_agg(B, Nn, De)
    M = sc_call(A, E.reshape(B, Nn, Nn * De))

    H4 = H.reshape(B * Nn, D)
    A4 = A.reshape(B * Nn, Nn)
    M4 = M.reshape(B * Nn, De)
    W_cat = jnp.concatenate([W_self, W_nei[:D]], axis=1)
    W_e = W_nei[D:]
    bias2 = bias[None, :]

    RB = 512
    grid = (B * Nn // RB,)
    out = pl.pallas_call(
        functools.partial(_tc_body, d=D),
        grid=grid,
        in_specs=[
            pl.BlockSpec((RB, D), lambda r: (r, 0)),
            pl.BlockSpec((RB, Nn), lambda r: (r, 0)),
            pl.BlockSpec((RB, De), lambda r: (r, 0)),
            pl.BlockSpec((D, 2 * D), lambda r: (0, 0)),
            pl.BlockSpec((De, D), lambda r: (0, 0)),
            pl.BlockSpec((1, D), lambda r: (0, 0)),
        ],
        out_specs=pl.BlockSpec((RB, D), lambda r: (r, 0)),
        out_shape=jax.ShapeDtypeStruct((B * Nn, D), jnp.float32),
        compiler_params=pltpu.CompilerParams(
            dimension_semantics=("arbitrary",),
        ),
    )(H4, A4, M4, W_cat, W_e, bias2)
    return out.reshape(B, Nn, D)
